# Initial kernel scaffold; baseline (speedup 1.0000x reference)
import jax
import jax.numpy as jnp
from jax.experimental import pallas as pl


def kernel(x, edge_index, batch, w1a, b1a, w1b, b1b, w2a, b2a, w2b, b2b, w3a, b3a, w3b, b3b, w4a, b4a, w4b, b4b, w5a, b5a, w5b, b5b, l1, l2, l3, l4, l5):
    raise NotImplementedError("write your pallas kernel here")


# speedup vs baseline: 4.4743x; 4.4743x over previous
import functools

import jax
import jax.numpy as jnp
from jax import lax
from jax.experimental import pallas as pl
from jax.experimental.pallas import tpu as pltpu
from jax.experimental.pallas import tpu_sc as plsc

_N = 10000
_E = 320000
_DIM = 64
_G = 128
_NW = 32
_K = 128
_NCHUNK = 80
_EPAD = _NW * _NCHUNK * _K
_ROWS_PER_TILE = _N // 16
_AGG_ROWS = _N + 16


def _sc_scatter_body(zeros_hbm, y_hbm, src_hbm, dst_hbm, out_hbm,
                     src_idx, dst_idx, rows, agg_sh, sem_g):
    c = lax.axis_index("c")
    s = lax.axis_index("s")
    wid = c * 16 + s

    pltpu.sync_copy(src_hbm.at[wid], src_idx)
    pltpu.sync_copy(dst_hbm.at[wid], dst_idx)

    pltpu.sync_copy(zeros_hbm.at[pl.ds(s * _ROWS_PER_TILE, _ROWS_PER_TILE)],
                    agg_sh.at[pl.ds(s * _ROWS_PER_TILE, _ROWS_PER_TILE)])
    plsc.subcore_barrier()

    def step(j, carry):
        pltpu.async_copy(y_hbm.at[src_idx.at[j]], rows, sem_g).wait()
        pltpu.sync_copy(rows, agg_sh.at[dst_idx.at[j]], add=True)
        return carry

    lax.fori_loop(0, _NCHUNK, step, 0, unroll=False)
    plsc.subcore_barrier()

    pltpu.sync_copy(agg_sh.at[pl.ds(s * _ROWS_PER_TILE, _ROWS_PER_TILE)],
                    out_hbm.at[c, pl.ds(s * _ROWS_PER_TILE, _ROWS_PER_TILE)])


_sc_scatter = pl.kernel(
    _sc_scatter_body,
    out_type=jax.ShapeDtypeStruct((2, _N, _DIM), jnp.float32),
    mesh=plsc.VectorSubcoreMesh(core_axis_name="c", subcore_axis_name="s"),
    scratch_types=[
        pltpu.VMEM((_NCHUNK, _K), jnp.int32),
        pltpu.VMEM((_NCHUNK, _K), jnp.int32),
        pltpu.VMEM((_K, _DIM), jnp.float32),
        pltpu.VMEM_SHARED((_AGG_ROWS, _DIM), jnp.float32),
        pltpu.SemaphoreType.DMA,
    ],
    compiler_params=pltpu.CompilerParams(use_tc_tiling_on_sc=False),
)


def _proj_body(x_ref, w_ref, o_ref):
    o_ref[...] = jnp.dot(x_ref[...], w_ref[...],
                         preferred_element_type=jnp.float32)


def _layer_body(y_ref, parts_ref, ba_ref, wb_ref, bb_ref, wan_ref, ll_ref,
                p_ref, ynext_ref, pout_ref):
    h = y_ref[...] + parts_ref[0] + parts_ref[1] + ba_ref[...]
    h = jnp.maximum(h, 0.0)
    xo = jnp.dot(h, wb_ref[...], preferred_element_type=jnp.float32)
    xo = jnp.maximum(xo + bb_ref[...], 0.0)
    ynext_ref[...] = jnp.dot(xo, wan_ref[...],
                             preferred_element_type=jnp.float32)
    pout_ref[...] = p_ref[...] + jnp.dot(xo, ll_ref[...],
                                         preferred_element_type=jnp.float32)


def _last_layer_body(y_ref, parts_ref, ba_ref, wb_ref, bb_ref, ll_ref,
                     p_ref, pout_ref):
    h = y_ref[...] + parts_ref[0] + parts_ref[1] + ba_ref[...]
    h = jnp.maximum(h, 0.0)
    xo = jnp.dot(h, wb_ref[...], preferred_element_type=jnp.float32)
    xo = jnp.maximum(xo + bb_ref[...], 0.0)
    pout_ref[...] = p_ref[...] + jnp.dot(xo, ll_ref[...],
                                         preferred_element_type=jnp.float32)


def _pool_body(batch_ref, p_ref, o_ref):
    gid = lax.broadcasted_iota(jnp.int32, (_G, _N), 0)
    mask = (gid == batch_ref[...]).astype(jnp.float32)
    s = jnp.dot(mask, p_ref[...], preferred_element_type=jnp.float32)
    cnt = jnp.sum(mask, axis=1, keepdims=True)
    o_ref[...] = jax.nn.sigmoid(s / jnp.maximum(cnt, 1.0))


_proj = pl.pallas_call(
    _proj_body,
    out_shape=jax.ShapeDtypeStruct((_N, _DIM), jnp.float32),
)

_layer = pl.pallas_call(
    _layer_body,
    out_shape=(jax.ShapeDtypeStruct((_N, _DIM), jnp.float32),
               jax.ShapeDtypeStruct((_N, 1), jnp.float32)),
)

_last_layer = pl.pallas_call(
    _last_layer_body,
    out_shape=jax.ShapeDtypeStruct((_N, 1), jnp.float32),
)

_pool = pl.pallas_call(
    _pool_body,
    out_shape=jax.ShapeDtypeStruct((_G, 1), jnp.float32),
)


def kernel(x, edge_index, batch, w1a, b1a, w1b, b1b, w2a, b2a, w2b, b2b,
           w3a, b3a, w3b, b3b, w4a, b4a, w4b, b4b, w5a, b5a, w5b, b5b,
           l1, l2, l3, l4, l5):
    src = edge_index[0]
    dst = edge_index[1]
    pad = _EPAD - _E
    src_p = jnp.concatenate(
        [src, jnp.zeros((pad,), jnp.int32)]).reshape(_NW, _NCHUNK, _K)
    dst_p = jnp.concatenate(
        [dst, jnp.full((pad,), _N, jnp.int32)]).reshape(_NW, _NCHUNK, _K)
    zeros_tab = jnp.zeros((_N, _DIM), jnp.float32)
    batch2d = batch.reshape(1, _N)

    was = [w1a, w2a, w3a, w4a, w5a]
    bas = [b1a.reshape(1, _DIM), b2a.reshape(1, _DIM), b3a.reshape(1, _DIM),
           b4a.reshape(1, _DIM), b5a.reshape(1, _DIM)]
    wbs = [w1b, w2b, w3b, w4b, w5b]
    bbs = [b1b.reshape(1, _DIM), b2b.reshape(1, _DIM), b3b.reshape(1, _DIM),
           b4b.reshape(1, _DIM), b5b.reshape(1, _DIM)]
    lls = [l1, l2, l3, l4, l5]

    y = _proj(x, w1a)
    p = jnp.zeros((_N, 1), jnp.float32)
    for i in range(5):
        parts = _sc_scatter(zeros_tab, y, src_p, dst_p)
        if i < 4:
            y, p = _layer(y, parts, bas[i], wbs[i], bbs[i], was[i + 1],
                          lls[i], p)
        else:
            p = _last_layer(y, parts, bas[i], wbs[i], bbs[i], lls[i], p)
    return _pool(batch2d, p)


# speedup vs baseline: 5.2715x; 1.1782x over previous
import functools

import jax
import jax.numpy as jnp
from jax import lax
from jax.experimental import pallas as pl
from jax.experimental.pallas import tpu as pltpu
from jax.experimental.pallas import tpu_sc as plsc

_N = 10000
_E = 320000
_DIM = 64
_G = 128
_NW = 32
_K = 128
_NCHUNK = 80
_EPAD = _NW * _NCHUNK * _K
_ROWS_PER_TILE = _N // 16
_AGG_ROWS = _N + 16


_NBUF = 8
_NGRP = _NCHUNK // _NBUF


def _sc_scatter_body(zeros_hbm, y_hbm, src_hbm, dst_hbm, out_hbm,
                     src_idx, dst_idx, rows, agg_sh, sem_g, sem_s):
    c = lax.axis_index("c")
    s = lax.axis_index("s")
    wid = c * 16 + s

    pltpu.sync_copy(src_hbm.at[wid], src_idx)
    pltpu.sync_copy(dst_hbm.at[wid], dst_idx)

    pltpu.sync_copy(zeros_hbm.at[pl.ds(s * _ROWS_PER_TILE, _ROWS_PER_TILE)],
                    agg_sh.at[pl.ds(s * _ROWS_PER_TILE, _ROWS_PER_TILE)])
    plsc.subcore_barrier()

    def _gather(j, b):
        return pltpu.make_async_copy(y_hbm.at[src_idx.at[j]], rows.at[b],
                                     sem_g)

    def _scatter(j, b):
        return pltpu.make_async_copy(rows.at[b], agg_sh.at[dst_idx.at[j]],
                                     sem_s)

    for b in range(_NBUF):
        _gather(b, b).start()

    def group(t, carry):
        base = t * _NBUF
        for b in range(_NBUF):
            _gather(base + b, b).wait()
            _scatter(base + b, b).start(add=True)
        for b in range(_NBUF):
            _scatter(base + b, b).wait()
            _gather(base + _NBUF + b, b).start()
        return carry

    lax.fori_loop(0, _NGRP - 1, group, 0, unroll=False)
    base = (_NGRP - 1) * _NBUF
    for b in range(_NBUF):
        _gather(base + b, b).wait()
        _scatter(base + b, b).start(add=True)
    for b in range(_NBUF):
        _scatter(base + b, b).wait()
    plsc.subcore_barrier()

    pltpu.sync_copy(agg_sh.at[pl.ds(s * _ROWS_PER_TILE, _ROWS_PER_TILE)],
                    out_hbm.at[c, pl.ds(s * _ROWS_PER_TILE, _ROWS_PER_TILE)])


_sc_scatter = pl.kernel(
    _sc_scatter_body,
    out_type=jax.ShapeDtypeStruct((2, _N, _DIM), jnp.float32),
    mesh=plsc.VectorSubcoreMesh(core_axis_name="c", subcore_axis_name="s"),
    scratch_types=[
        pltpu.VMEM((_NCHUNK, _K), jnp.int32),
        pltpu.VMEM((_NCHUNK, _K), jnp.int32),
        pltpu.VMEM((_NBUF, _K, _DIM), jnp.float32),
        pltpu.VMEM_SHARED((_AGG_ROWS, _DIM), jnp.float32),
        pltpu.SemaphoreType.DMA,
        pltpu.SemaphoreType.DMA,
    ],
    compiler_params=pltpu.CompilerParams(use_tc_tiling_on_sc=False),
)


def _proj_body(x_ref, w_ref, o_ref):
    o_ref[...] = jnp.dot(x_ref[...], w_ref[...],
                         preferred_element_type=jnp.float32)


def _layer_body(y_ref, parts_ref, ba_ref, wb_ref, bb_ref, wan_ref, ll_ref,
                p_ref, ynext_ref, pout_ref):
    h = y_ref[...] + parts_ref[0] + parts_ref[1] + ba_ref[...]
    h = jnp.maximum(h, 0.0)
    xo = jnp.dot(h, wb_ref[...], preferred_element_type=jnp.float32)
    xo = jnp.maximum(xo + bb_ref[...], 0.0)
    ynext_ref[...] = jnp.dot(xo, wan_ref[...],
                             preferred_element_type=jnp.float32)
    pout_ref[...] = p_ref[...] + jnp.dot(xo, ll_ref[...],
                                         preferred_element_type=jnp.float32)


def _last_layer_body(y_ref, parts_ref, ba_ref, wb_ref, bb_ref, ll_ref,
                     p_ref, pout_ref):
    h = y_ref[...] + parts_ref[0] + parts_ref[1] + ba_ref[...]
    h = jnp.maximum(h, 0.0)
    xo = jnp.dot(h, wb_ref[...], preferred_element_type=jnp.float32)
    xo = jnp.maximum(xo + bb_ref[...], 0.0)
    pout_ref[...] = p_ref[...] + jnp.dot(xo, ll_ref[...],
                                         preferred_element_type=jnp.float32)


def _pool_body(batch_ref, p_ref, o_ref):
    gid = lax.broadcasted_iota(jnp.int32, (_G, _N), 0)
    mask = (gid == batch_ref[...]).astype(jnp.float32)
    s = jnp.dot(mask, p_ref[...], preferred_element_type=jnp.float32)
    cnt = jnp.sum(mask, axis=1, keepdims=True)
    o_ref[...] = jax.nn.sigmoid(s / jnp.maximum(cnt, 1.0))


_proj = pl.pallas_call(
    _proj_body,
    out_shape=jax.ShapeDtypeStruct((_N, _DIM), jnp.float32),
)

_layer = pl.pallas_call(
    _layer_body,
    out_shape=(jax.ShapeDtypeStruct((_N, _DIM), jnp.float32),
               jax.ShapeDtypeStruct((_N, 1), jnp.float32)),
)

_last_layer = pl.pallas_call(
    _last_layer_body,
    out_shape=jax.ShapeDtypeStruct((_N, 1), jnp.float32),
)

_pool = pl.pallas_call(
    _pool_body,
    out_shape=jax.ShapeDtypeStruct((_G, 1), jnp.float32),
)


def kernel(x, edge_index, batch, w1a, b1a, w1b, b1b, w2a, b2a, w2b, b2b,
           w3a, b3a, w3b, b3b, w4a, b4a, w4b, b4b, w5a, b5a, w5b, b5b,
           l1, l2, l3, l4, l5):
    src = edge_index[0]
    dst = edge_index[1]
    pad = _EPAD - _E
    src_p = jnp.concatenate(
        [src, jnp.zeros((pad,), jnp.int32)]).reshape(_NW, _NCHUNK, _K)
    dst_p = jnp.concatenate(
        [dst, jnp.full((pad,), _N, jnp.int32)]).reshape(_NW, _NCHUNK, _K)
    zeros_tab = jnp.zeros((_N, _DIM), jnp.float32)
    batch2d = batch.reshape(1, _N)

    was = [w1a, w2a, w3a, w4a, w5a]
    bas = [b1a.reshape(1, _DIM), b2a.reshape(1, _DIM), b3a.reshape(1, _DIM),
           b4a.reshape(1, _DIM), b5a.reshape(1, _DIM)]
    wbs = [w1b, w2b, w3b, w4b, w5b]
    bbs = [b1b.reshape(1, _DIM), b2b.reshape(1, _DIM), b3b.reshape(1, _DIM),
           b4b.reshape(1, _DIM), b5b.reshape(1, _DIM)]
    lls = [l1, l2, l3, l4, l5]

    y = _proj(x, w1a)
    p = jnp.zeros((_N, 1), jnp.float32)
    for i in range(5):
        parts = _sc_scatter(zeros_tab, y, src_p, dst_p)
        if i < 4:
            y, p = _layer(y, parts, bas[i], wbs[i], bbs[i], was[i + 1],
                          lls[i], p)
        else:
            p = _last_layer(y, parts, bas[i], wbs[i], bbs[i], lls[i], p)
    return _pool(batch2d, p)
